# baseline (speedup 1.0000x reference)
import jax
import jax.numpy as jnp
from jax.experimental import pallas as pl
from jax.experimental.pallas import tpu as pltpu

HIDDEN = 4096
N_EXPERTS = 128
TOKENS_TOTAL = 16384
C = 256
D = 8
NCHUNK = TOKENS_TOTAL // C


def _router_kernel(hs_ref, w_ref, out_ref, in_buf, out_buf, in_sem, out_sem):
    w_bf = w_ref[...]

    def in_copy(i, slot):
        return pltpu.make_async_copy(
            hs_ref.at[pl.ds(i * C, C), :], in_buf.at[slot], in_sem.at[slot])

    def out_copy(i, slot):
        return pltpu.make_async_copy(
            out_buf.at[slot], out_ref.at[pl.ds(i * C, C), :], out_sem.at[slot])

    for j in range(D):
        in_copy(j, j).start()

    for i in range(NCHUNK):
        slot = i % D
        in_copy(i, slot).wait()
        if i >= D:
            out_copy(i - D, slot).wait()
        out_buf[slot] = jax.lax.dot_general(
            in_buf[slot].astype(jnp.bfloat16),
            w_bf,
            dimension_numbers=(((1,), (1,)), ((), ())),
            preferred_element_type=jnp.float32,
        )
        out_copy(i, slot).start()
        if i + D < NCHUNK:
            in_copy(i + D, slot).start()

    for i in range(NCHUNK - D, NCHUNK):
        out_copy(i, i % D).wait()


def kernel(hidden_states, W):
    hs = hidden_states.reshape(-1, HIDDEN).astype(jnp.float32)
    m = hs.shape[0]
    return pl.pallas_call(
        _router_kernel,
        in_specs=[
            pl.BlockSpec(memory_space=pltpu.HBM),
            pl.BlockSpec(memory_space=pltpu.VMEM),
        ],
        out_specs=pl.BlockSpec(memory_space=pltpu.HBM),
        out_shape=jax.ShapeDtypeStruct((m, N_EXPERTS), jnp.float32),
        scratch_shapes=[
            pltpu.VMEM((D, C, HIDDEN), jnp.float32),
            pltpu.VMEM((D, C, N_EXPERTS), jnp.float32),
            pltpu.SemaphoreType.DMA((D,)),
            pltpu.SemaphoreType.DMA((D,)),
        ],
    )(hs, W.astype(jnp.bfloat16))


# speedup vs baseline: 1.1005x; 1.1005x over previous
import jax
import jax.numpy as jnp
from jax.experimental import pallas as pl
from jax.experimental.pallas import tpu as pltpu

HIDDEN = 4096
N_EXPERTS = 128
BM = 512


def _router_kernel(hs_ref, w_ref, out_ref):
    out_ref[...] = jax.lax.dot_general(
        hs_ref[...].astype(jnp.bfloat16),
        w_ref[...],
        dimension_numbers=(((1,), (1,)), ((), ())),
        preferred_element_type=jnp.float32,
    )


def kernel(hidden_states, W):
    hs = hidden_states.reshape(-1, HIDDEN).astype(jnp.float32)
    m = hs.shape[0]
    grid = (m // BM,)
    return pl.pallas_call(
        _router_kernel,
        grid=grid,
        in_specs=[
            pl.BlockSpec((BM, HIDDEN), lambda i: (i, 0)),
            pl.BlockSpec((N_EXPERTS, HIDDEN), lambda i: (0, 0)),
        ],
        out_specs=pl.BlockSpec((BM, N_EXPERTS), lambda i: (i, 0)),
        out_shape=jax.ShapeDtypeStruct((m, N_EXPERTS), jnp.float32),
        compiler_params=pltpu.CompilerParams(
            dimension_semantics=("parallel",),
        ),
    )(hs, W.astype(jnp.bfloat16))


# speedup vs baseline: 1.1298x; 1.0266x over previous
import jax
import jax.numpy as jnp
from jax.experimental import pallas as pl
from jax.experimental.pallas import tpu as pltpu

HIDDEN = 4096
N_EXPERTS = 128
BM = 512


def _router_kernel(hs_ref, w_ref, out_ref):
    out_ref[...] = jax.lax.dot_general(
        hs_ref[...].astype(jnp.bfloat16),
        w_ref[...].astype(jnp.bfloat16),
        dimension_numbers=(((1,), (1,)), ((), ())),
        preferred_element_type=jnp.float32,
    )


def kernel(hidden_states, W):
    hs = hidden_states.reshape(-1, HIDDEN).astype(jnp.float32)
    m = hs.shape[0]
    grid = (m // BM,)
    return pl.pallas_call(
        _router_kernel,
        grid=grid,
        in_specs=[
            pl.BlockSpec((BM, HIDDEN), lambda i: (i, 0)),
            pl.BlockSpec((N_EXPERTS, HIDDEN), lambda i: (0, 0)),
        ],
        out_specs=pl.BlockSpec((BM, N_EXPERTS), lambda i: (i, 0)),
        out_shape=jax.ShapeDtypeStruct((m, N_EXPERTS), jnp.float32),
        compiler_params=pltpu.CompilerParams(
            dimension_semantics=("parallel",),
        ),
    )(hs, W)
